# Initial kernel scaffold; baseline (speedup 1.0000x reference)
import jax
import jax.numpy as jnp
from jax.experimental import pallas as pl


def kernel(z, embedding, device):
    raise NotImplementedError("write your pallas kernel here")


# speedup vs baseline: 1.1077x; 1.1077x over previous
import jax
import jax.numpy as jnp
from jax.experimental import pallas as pl
from jax.experimental.pallas import tpu as pltpu

N_SPLIT_K = 4
N_CODES = 8192
EDIM = 256
ROWS = 4096
TILE = 256
NUM_TILES = ROWS // TILE


def _vq_body(z_ref, et_ref, e_ref,
             zq_ref, oh_ref, idx_ref, loss_ref, mse_ref, ppl_ref,
             hist_ref, sse_ref):
    i = pl.program_id(0)

    @pl.when(i == 0)
    def _init():
        hist_ref[...] = jnp.zeros_like(hist_ref)
        sse_ref[...] = jnp.zeros_like(sse_ref)

    zt = z_ref[...]
    et = et_ref[...]
    z_sq = jnp.sum(zt * zt, axis=1, keepdims=True)
    e_sq = jnp.sum(et * et, axis=0, keepdims=True)
    mm = jnp.dot(zt, et, preferred_element_type=jnp.float32)
    d = (z_sq + e_sq) - 2.0 * mm

    iota = jax.lax.broadcasted_iota(jnp.int32, d.shape, 1)
    m = jnp.min(d, axis=1, keepdims=True)
    idx = jnp.min(jnp.where(d == m, iota, jnp.int32(N_CODES)),
                  axis=1, keepdims=True)
    oh = (iota == idx).astype(jnp.float32)
    zq = jnp.dot(oh, e_ref[...], preferred_element_type=jnp.float32)

    zq_ref[...] = zq
    oh_ref[...] = oh
    idx_ref[...] = idx

    diff = zq - zt
    sse_ref[...] += jnp.sum(diff * diff, axis=(0, 1), keepdims=True)
    hist_ref[...] += jnp.sum(oh, axis=0, keepdims=True)

    @pl.when(i == NUM_TILES - 1)
    def _finish():
        mse = sse_ref[...] * jnp.float32(1.0 / ROWS / EDIM)
        mse_ref[...] = mse
        loss_ref[...] = mse + 0.25 * mse
        e_mean = hist_ref[...] * jnp.float32(1.0 / ROWS)
        ent = -jnp.sum(e_mean * jnp.log(e_mean + 1e-10),
                       axis=(0, 1), keepdims=True)
        ppl_ref[...] = jnp.exp(ent)


def kernel(z, embedding, device):
    z_flat = z.reshape(-1, EDIM)
    emb_t = embedding.T

    zq, oh, idx, loss, mse, ppl = pl.pallas_call(
        _vq_body,
        grid=(NUM_TILES,),
        in_specs=[
            pl.BlockSpec((TILE, EDIM), lambda i: (i, 0)),
            pl.BlockSpec((EDIM, N_CODES), lambda i: (0, 0)),
            pl.BlockSpec((N_CODES, EDIM), lambda i: (0, 0)),
        ],
        out_specs=[
            pl.BlockSpec((TILE, EDIM), lambda i: (i, 0)),
            pl.BlockSpec((TILE, N_CODES), lambda i: (i, 0)),
            pl.BlockSpec((TILE, 1), lambda i: (i, 0)),
            pl.BlockSpec((1, 1), lambda i: (0, 0)),
            pl.BlockSpec((1, 1), lambda i: (0, 0)),
            pl.BlockSpec((1, 1), lambda i: (0, 0)),
        ],
        out_shape=[
            jax.ShapeDtypeStruct((ROWS, EDIM), jnp.float32),
            jax.ShapeDtypeStruct((ROWS, N_CODES), jnp.float32),
            jax.ShapeDtypeStruct((ROWS, 1), jnp.int32),
            jax.ShapeDtypeStruct((1, 1), jnp.float32),
            jax.ShapeDtypeStruct((1, 1), jnp.float32),
            jax.ShapeDtypeStruct((1, 1), jnp.float32),
        ],
        scratch_shapes=[
            pltpu.VMEM((1, N_CODES), jnp.float32),
            pltpu.VMEM((1, 1), jnp.float32),
        ],
    )(z_flat, emb_t, embedding)

    loss_s = loss[0, 0]
    mse_s = mse[0, 0]
    z_q_st = zq.reshape(z.shape[0], N_SPLIT_K, EDIM)
    z_output = zq.reshape(z.shape[0], -1)
    return (loss_s, mse_s, mse_s, z_q_st, z_output, ppl[0, 0], oh, idx)


# speedup vs baseline: 1.3515x; 1.2201x over previous
import jax
import jax.numpy as jnp
from jax.experimental import pallas as pl
from jax.experimental.pallas import tpu as pltpu

N_SPLIT_K = 4
N_CODES = 8192
EDIM = 256
BATCH = 1024
ROWS = BATCH * N_SPLIT_K
TILE = 256
BTILE = TILE // N_SPLIT_K
NUM_TILES = ROWS // TILE

_RHS_T = (((1,), (1,)), ((), ()))


def _vq_body(z_ref, e_ref,
             zqst_ref, zout_ref, oh_ref, idx_ref, loss_ref, mse_ref, ppl_ref,
             esq_ref, hist_ref, sse_ref):
    i = pl.program_id(0)
    e = e_ref[...]

    @pl.when(i == 0)
    def _init():
        hist_ref[...] = jnp.zeros_like(hist_ref)
        sse_ref[...] = jnp.zeros_like(sse_ref)
        sq = e * e
        ones = jnp.ones((1, EDIM), jnp.float32)
        esq_ref[...] = jax.lax.dot_general(
            ones, sq, _RHS_T, preferred_element_type=jnp.float32)

    zt = z_ref[...].reshape(TILE, EDIM)
    z_sq = jnp.sum(zt * zt, axis=1, keepdims=True)
    e_sq = esq_ref[...]
    mm = jax.lax.dot_general(zt, e, _RHS_T,
                             preferred_element_type=jnp.float32)
    d = (z_sq + e_sq) - 2.0 * mm

    iota = jax.lax.broadcasted_iota(jnp.int32, d.shape, 1)
    m = jnp.min(d, axis=1, keepdims=True)
    idx = jnp.min(jnp.where(d == m, iota, jnp.int32(N_CODES)),
                  axis=1, keepdims=True)
    oh = (iota == idx).astype(jnp.float32)
    zq = jnp.dot(oh, e, preferred_element_type=jnp.float32)

    zqst_ref[...] = zq.reshape(BTILE, N_SPLIT_K, EDIM)
    zout_ref[...] = zq.reshape(BTILE, N_SPLIT_K * EDIM)
    oh_ref[...] = oh
    idx_ref[...] = idx

    diff = zq - zt
    sse_ref[...] += jnp.sum(diff * diff, axis=(0, 1), keepdims=True)
    hist_ref[...] += jnp.sum(oh, axis=0, keepdims=True)

    @pl.when(i == NUM_TILES - 1)
    def _finish():
        mse = sse_ref[...] * jnp.float32(1.0 / ROWS / EDIM)
        mse_ref[...] = mse
        loss_ref[...] = mse + 0.25 * mse
        e_mean = hist_ref[...] * jnp.float32(1.0 / ROWS)
        ent = -jnp.sum(e_mean * jnp.log(e_mean + 1e-10),
                       axis=(0, 1), keepdims=True)
        ppl_ref[...] = jnp.exp(ent)


def kernel(z, embedding, device):
    zqst, zout, oh, idx, loss, mse, ppl = pl.pallas_call(
        _vq_body,
        grid=(NUM_TILES,),
        in_specs=[
            pl.BlockSpec((BTILE, N_SPLIT_K * EDIM), lambda i: (i, 0)),
            pl.BlockSpec((N_CODES, EDIM), lambda i: (0, 0)),
        ],
        out_specs=[
            pl.BlockSpec((BTILE, N_SPLIT_K, EDIM), lambda i: (i, 0, 0)),
            pl.BlockSpec((BTILE, N_SPLIT_K * EDIM), lambda i: (i, 0)),
            pl.BlockSpec((TILE, N_CODES), lambda i: (i, 0)),
            pl.BlockSpec((TILE, 1), lambda i: (i, 0)),
            pl.BlockSpec((1, 1), lambda i: (0, 0)),
            pl.BlockSpec((1, 1), lambda i: (0, 0)),
            pl.BlockSpec((1, 1), lambda i: (0, 0)),
        ],
        out_shape=[
            jax.ShapeDtypeStruct((BATCH, N_SPLIT_K, EDIM), jnp.float32),
            jax.ShapeDtypeStruct((BATCH, N_SPLIT_K * EDIM), jnp.float32),
            jax.ShapeDtypeStruct((ROWS, N_CODES), jnp.float32),
            jax.ShapeDtypeStruct((ROWS, 1), jnp.int32),
            jax.ShapeDtypeStruct((1, 1), jnp.float32),
            jax.ShapeDtypeStruct((1, 1), jnp.float32),
            jax.ShapeDtypeStruct((1, 1), jnp.float32),
        ],
        scratch_shapes=[
            pltpu.VMEM((1, N_CODES), jnp.float32),
            pltpu.VMEM((1, N_CODES), jnp.float32),
            pltpu.VMEM((1, 1), jnp.float32),
        ],
    )(z, embedding)

    return (loss[0, 0], mse[0, 0], mse[0, 0], zqst, zout, ppl[0, 0], oh, idx)


# speedup vs baseline: 1.4376x; 1.0637x over previous
import jax
import jax.numpy as jnp
from jax.experimental import pallas as pl
from jax.experimental.pallas import tpu as pltpu

N_SPLIT_K = 4
N_CODES = 8192
EDIM = 256
BATCH = 1024
ROWS = BATCH * N_SPLIT_K
TILE = 256
BTILE = TILE // N_SPLIT_K
NUM_TILES = ROWS // TILE

_RHS_T = (((1,), (1,)), ((), ()))


def _vq_body(z_ref, e_ref,
             zqst_ref, zout_ref, oh_ref, idx_ref, loss_ref, mse_ref, ppl_ref,
             esq_ref, hist_ref, sse_ref):
    i = pl.program_id(0)
    e = e_ref[...]

    @pl.when(i == 0)
    def _init():
        hist_ref[...] = jnp.zeros_like(hist_ref)
        sse_ref[...] = jnp.zeros_like(sse_ref)
        sq = e * e
        ones = jnp.ones((1, EDIM), jnp.float32)
        esq_ref[...] = jax.lax.dot_general(
            ones, sq, _RHS_T, preferred_element_type=jnp.float32)

    zt = z_ref[...].reshape(TILE, EDIM)
    z_sq = jnp.sum(zt * zt, axis=1, keepdims=True)
    e_sq = esq_ref[...]
    mm = jax.lax.dot_general(zt, e, _RHS_T,
                             preferred_element_type=jnp.float32)
    d = (z_sq + e_sq) - 2.0 * mm

    iota = jax.lax.broadcasted_iota(jnp.int32, d.shape, 1)
    m = jnp.min(d, axis=1, keepdims=True)
    idx = jnp.min(jnp.where(d == m, iota, jnp.int32(N_CODES)),
                  axis=1, keepdims=True)
    oh = (iota == idx).astype(jnp.float32)
    zq = jnp.dot(oh, e, preferred_element_type=jnp.float32)

    zqst_ref[...] = zq.reshape(BTILE, N_SPLIT_K, EDIM)
    zout_ref[...] = zq.reshape(BTILE, N_SPLIT_K * EDIM)
    oh_ref[...] = oh
    idx_ref[...] = idx

    diff = zq - zt
    sse_ref[...] += jnp.sum(diff * diff, axis=(0, 1), keepdims=True)
    ones_r = jnp.ones((1, TILE), jnp.float32)
    hist_ref[...] += jax.lax.dot_general(
        ones_r, oh, (((1,), (0,)), ((), ())),
        preferred_element_type=jnp.float32)

    @pl.when(i == NUM_TILES - 1)
    def _finish():
        mse = sse_ref[...] * jnp.float32(1.0 / ROWS / EDIM)
        mse_ref[...] = mse
        loss_ref[...] = mse + 0.25 * mse
        e_mean = hist_ref[...] * jnp.float32(1.0 / ROWS)
        ent = -jnp.sum(e_mean * jnp.log(e_mean + 1e-10),
                       axis=(0, 1), keepdims=True)
        ppl_ref[...] = jnp.exp(ent)


def kernel(z, embedding, device):
    zqst, zout, oh, idx, loss, mse, ppl = pl.pallas_call(
        _vq_body,
        grid=(NUM_TILES,),
        in_specs=[
            pl.BlockSpec((BTILE, N_SPLIT_K * EDIM), lambda i: (i, 0)),
            pl.BlockSpec((N_CODES, EDIM), lambda i: (0, 0)),
        ],
        out_specs=[
            pl.BlockSpec((BTILE, N_SPLIT_K, EDIM), lambda i: (i, 0, 0)),
            pl.BlockSpec((BTILE, N_SPLIT_K * EDIM), lambda i: (i, 0)),
            pl.BlockSpec((TILE, N_CODES), lambda i: (i, 0)),
            pl.BlockSpec((TILE, 1), lambda i: (i, 0)),
            pl.BlockSpec((1, 1), lambda i: (0, 0)),
            pl.BlockSpec((1, 1), lambda i: (0, 0)),
            pl.BlockSpec((1, 1), lambda i: (0, 0)),
        ],
        out_shape=[
            jax.ShapeDtypeStruct((BATCH, N_SPLIT_K, EDIM), jnp.float32),
            jax.ShapeDtypeStruct((BATCH, N_SPLIT_K * EDIM), jnp.float32),
            jax.ShapeDtypeStruct((ROWS, N_CODES), jnp.float32),
            jax.ShapeDtypeStruct((ROWS, 1), jnp.int32),
            jax.ShapeDtypeStruct((1, 1), jnp.float32),
            jax.ShapeDtypeStruct((1, 1), jnp.float32),
            jax.ShapeDtypeStruct((1, 1), jnp.float32),
        ],
        scratch_shapes=[
            pltpu.VMEM((1, N_CODES), jnp.float32),
            pltpu.VMEM((1, N_CODES), jnp.float32),
            pltpu.VMEM((1, 1), jnp.float32),
        ],
    )(z, embedding)

    return (loss[0, 0], mse[0, 0], mse[0, 0], zqst, zout, ppl[0, 0], oh, idx)


# speedup vs baseline: 1.5324x; 1.0659x over previous
import jax
import jax.numpy as jnp
from jax.experimental import pallas as pl
from jax.experimental.pallas import tpu as pltpu

N_SPLIT_K = 4
N_CODES = 8192
EDIM = 256
BATCH = 1024
ROWS = BATCH * N_SPLIT_K
TILE = 256
BTILE = TILE // N_SPLIT_K
NUM_TILES = ROWS // TILE

_RHS_T = (((1,), (1,)), ((), ()))


def _vq_body(z_ref, e_ref,
             zqst_ref, zout_ref, oh_ref, idx_ref, loss_ref, mse_ref, ppl_ref,
             esq_ref, hist_ref, sse_ref):
    i = pl.program_id(0)

    @pl.when(i == 0)
    def _init():
        hist_ref[...] = jnp.zeros_like(hist_ref)
        sse_ref[...] = jnp.zeros_like(sse_ref)
        sq = e_ref[...] * e_ref[...]
        ones = jnp.ones((1, EDIM), jnp.float32)
        esq_ref[...] = jax.lax.dot_general(
            ones, sq, _RHS_T, preferred_element_type=jnp.float32)

    zt = z_ref[...].reshape(TILE, EDIM)
    z_sq = jnp.sum(zt * zt, axis=1, keepdims=True)
    e_sq = esq_ref[...]
    mm = jax.lax.dot_general(zt, e_ref[...], _RHS_T,
                             preferred_element_type=jnp.float32)
    d = (z_sq + e_sq) - 2.0 * mm

    iota = jax.lax.broadcasted_iota(jnp.int32, d.shape, 1)
    m = jnp.min(d, axis=1, keepdims=True)
    idx = jnp.min(jnp.where(d == m, iota, jnp.int32(N_CODES)),
                  axis=1, keepdims=True)
    oh = (iota == idx).astype(jnp.float32)
    zq = jnp.dot(oh, e_ref[...], preferred_element_type=jnp.float32)

    zqst_ref[...] = zq.reshape(BTILE, N_SPLIT_K, EDIM)
    zout_ref[...] = zq.reshape(BTILE, N_SPLIT_K * EDIM)
    oh_ref[...] = oh
    idx_ref[...] = idx

    diff = zq - zt
    sse_ref[...] += jnp.sum(diff * diff, axis=(0, 1), keepdims=True)
    ones_r = jnp.ones((1, TILE), jnp.float32)
    hist_ref[...] += jax.lax.dot_general(
        ones_r, oh, (((1,), (0,)), ((), ())),
        preferred_element_type=jnp.float32)

    @pl.when(i == NUM_TILES - 1)
    def _finish():
        mse = sse_ref[...] * jnp.float32(1.0 / ROWS / EDIM)
        mse_ref[...] = mse
        loss_ref[...] = mse + 0.25 * mse
        e_mean = hist_ref[...] * jnp.float32(1.0 / ROWS)
        ent = -jnp.sum(e_mean * jnp.log(e_mean + 1e-10),
                       axis=(0, 1), keepdims=True)
        ppl_ref[...] = jnp.exp(ent)


def kernel(z, embedding, device):
    zqst, zout, oh, idx, loss, mse, ppl = pl.pallas_call(
        _vq_body,
        grid=(NUM_TILES,),
        in_specs=[
            pl.BlockSpec((BTILE, N_SPLIT_K * EDIM), lambda i: (i, 0)),
            pl.BlockSpec((N_CODES, EDIM), lambda i: (0, 0)),
        ],
        out_specs=[
            pl.BlockSpec((BTILE, N_SPLIT_K, EDIM), lambda i: (i, 0, 0)),
            pl.BlockSpec((BTILE, N_SPLIT_K * EDIM), lambda i: (i, 0)),
            pl.BlockSpec((TILE, N_CODES), lambda i: (i, 0)),
            pl.BlockSpec((TILE, 1), lambda i: (i, 0)),
            pl.BlockSpec((1, 1), lambda i: (0, 0)),
            pl.BlockSpec((1, 1), lambda i: (0, 0)),
            pl.BlockSpec((1, 1), lambda i: (0, 0)),
        ],
        out_shape=[
            jax.ShapeDtypeStruct((BATCH, N_SPLIT_K, EDIM), jnp.float32),
            jax.ShapeDtypeStruct((BATCH, N_SPLIT_K * EDIM), jnp.float32),
            jax.ShapeDtypeStruct((ROWS, N_CODES), jnp.float32),
            jax.ShapeDtypeStruct((ROWS, 1), jnp.int32),
            jax.ShapeDtypeStruct((1, 1), jnp.float32),
            jax.ShapeDtypeStruct((1, 1), jnp.float32),
            jax.ShapeDtypeStruct((1, 1), jnp.float32),
        ],
        scratch_shapes=[
            pltpu.VMEM((1, N_CODES), jnp.float32),
            pltpu.VMEM((1, N_CODES), jnp.float32),
            pltpu.VMEM((1, 1), jnp.float32),
        ],
    )(z, embedding)

    return (loss[0, 0], mse[0, 0], mse[0, 0], zqst, zout, ppl[0, 0], oh, idx)


# speedup vs baseline: 1.5359x; 1.0023x over previous
import jax
import jax.numpy as jnp
from jax.experimental import pallas as pl
from jax.experimental.pallas import tpu as pltpu

N_SPLIT_K = 4
N_CODES = 8192
EDIM = 256
BATCH = 1024
ROWS = BATCH * N_SPLIT_K
TILE = 256
BTILE = TILE // N_SPLIT_K
NUM_TILES = ROWS // TILE

_RHS_T = (((1,), (1,)), ((), ()))


def _vq_body(z_ref, e_ref,
             zqst_ref, zout_ref, oh_ref, idx_ref, loss_ref, mse_ref, ppl_ref,
             acc_ref, sse_ref):
    i = pl.program_id(0)

    @pl.when(i == 0)
    def _init():
        acc_ref[1:2, :] = jnp.zeros((1, N_CODES), jnp.float32)
        sse_ref[...] = jnp.zeros_like(sse_ref)
        sq = e_ref[...] * e_ref[...]
        ones = jnp.ones((1, EDIM), jnp.float32)
        acc_ref[0:1, :] = jax.lax.dot_general(
            ones, sq, _RHS_T, preferred_element_type=jnp.float32)

    zt = z_ref[...].reshape(TILE, EDIM)
    z_sq = jnp.sum(zt * zt, axis=1, keepdims=True)
    e_sq = acc_ref[0:1, :]
    mm = jax.lax.dot_general(zt, e_ref[...], _RHS_T,
                             preferred_element_type=jnp.float32)
    d = (z_sq + e_sq) - 2.0 * mm

    iota = jax.lax.broadcasted_iota(jnp.int32, d.shape, 1)
    m = jnp.min(d, axis=1, keepdims=True)
    idx = jnp.min(jnp.where(d == m, iota, jnp.int32(N_CODES)),
                  axis=1, keepdims=True)
    oh = (iota == idx).astype(jnp.float32)
    zq = jnp.dot(oh, e_ref[...], preferred_element_type=jnp.float32)

    zqst_ref[...] = zq.reshape(BTILE, N_SPLIT_K, EDIM)
    zout_ref[...] = zq.reshape(BTILE, N_SPLIT_K * EDIM)
    oh_ref[...] = oh
    idx_ref[...] = idx

    diff = zq - zt
    sse_ref[...] += jnp.sum(diff * diff, axis=(0, 1), keepdims=True)
    ones_r = jnp.ones((1, TILE), jnp.float32)
    acc_ref[1:2, :] += jax.lax.dot_general(
        ones_r, oh, (((1,), (0,)), ((), ())),
        preferred_element_type=jnp.float32)

    @pl.when(i == NUM_TILES - 1)
    def _finish():
        mse = sse_ref[...] * jnp.float32(1.0 / ROWS / EDIM)
        mse_ref[...] = mse
        loss_ref[...] = mse + 0.25 * mse
        e_mean = acc_ref[1:2, :] * jnp.float32(1.0 / ROWS)
        ent = -jnp.sum(e_mean * jnp.log(e_mean + 1e-10),
                       axis=(0, 1), keepdims=True)
        ppl_ref[...] = jnp.exp(ent)


def kernel(z, embedding, device):
    zqst, zout, oh, idx, loss, mse, ppl = pl.pallas_call(
        _vq_body,
        grid=(NUM_TILES,),
        in_specs=[
            pl.BlockSpec((BTILE, N_SPLIT_K * EDIM), lambda i: (i, 0)),
            pl.BlockSpec((N_CODES, EDIM), lambda i: (0, 0)),
        ],
        out_specs=[
            pl.BlockSpec((BTILE, N_SPLIT_K, EDIM), lambda i: (i, 0, 0)),
            pl.BlockSpec((BTILE, N_SPLIT_K * EDIM), lambda i: (i, 0)),
            pl.BlockSpec((TILE, N_CODES), lambda i: (i, 0)),
            pl.BlockSpec((TILE, 1), lambda i: (i, 0)),
            pl.BlockSpec((1, 1), lambda i: (0, 0)),
            pl.BlockSpec((1, 1), lambda i: (0, 0)),
            pl.BlockSpec((1, 1), lambda i: (0, 0)),
        ],
        out_shape=[
            jax.ShapeDtypeStruct((BATCH, N_SPLIT_K, EDIM), jnp.float32),
            jax.ShapeDtypeStruct((BATCH, N_SPLIT_K * EDIM), jnp.float32),
            jax.ShapeDtypeStruct((ROWS, N_CODES), jnp.float32),
            jax.ShapeDtypeStruct((ROWS, 1), jnp.int32),
            jax.ShapeDtypeStruct((1, 1), jnp.float32),
            jax.ShapeDtypeStruct((1, 1), jnp.float32),
            jax.ShapeDtypeStruct((1, 1), jnp.float32),
        ],
        scratch_shapes=[
            pltpu.VMEM((2, N_CODES), jnp.float32),
            pltpu.VMEM((1, 1), jnp.float32),
        ],
    )(z, embedding)

    return (loss[0, 0], mse[0, 0], mse[0, 0], zqst, zout, ppl[0, 0], oh, idx)
